# Initial kernel scaffold; baseline (speedup 1.0000x reference)
import jax
import jax.numpy as jnp
from jax.experimental import pallas as pl


def kernel(s_feats, q_points, s_points, neighbor_indices, W1, gamma1, beta1, kernel_points, kp_weights, gamma_c, beta_c, W2, gamma2, beta2):
    raise NotImplementedError("write your pallas kernel here")


# speedup vs baseline: 1.0315x; 1.0315x over previous
import functools

import jax
import jax.numpy as jnp
import numpy as np
from jax import lax
from jax.experimental import pallas as pl
from jax.experimental.pallas import tpu as pltpu
from jax.experimental.pallas import tpu_sc as plsc

N = 10000
H = 16
IN_DIM = 256
OUT_DIM = 256
HID = 64
K = 15
SIGMA = 0.5
GROUPS = 8
EPS = 1e-5

TD = 128
MT = 400
NSTEP = N // MT
MT2 = 1000
NSTEP2 = N // MT2
TOT = N * H
TOTP = 163840


def _leaky(x):
    return jnp.where(x >= 0, x, 0.1 * x)


def _pass_a_body(sf_ref, w1_ref, g1_ref, b1_ref, sp_ref, p64_ref, t_ref):
    x = lax.dot_general(sf_ref[...], w1_ref[...],
                        (((1,), (1,)), ((), ())),
                        preferred_element_type=jnp.float32, precision=lax.Precision.HIGHEST)
    s0 = jnp.sum(x, axis=0, keepdims=True)
    s1 = jnp.sum(x * x, axis=0, keepdims=True)
    scale = 1.0 / (N * (HID // GROUPS))
    mu = (s0 @ p64_ref[...]) * scale
    ms = (s1 @ p64_ref[...]) * scale
    inv = lax.rsqrt(ms - mu * mu + EPS)
    x1 = _leaky((x - mu) * inv * g1_ref[...] + b1_ref[...])
    pad = jnp.zeros((N, TD - HID - 3), jnp.float32)
    t_ref[...] = jnp.concatenate([x1, sp_ref[...], pad], axis=1)


def _pass_a(s_feats, W1, g1, b1, s_points, p64):
    return pl.pallas_call(
        _pass_a_body,
        out_shape=jax.ShapeDtypeStruct((N, TD), jnp.float32),
    )(s_feats, W1, g1, b1, s_points, p64)


def _gather_rows(table, idx2d):
    info = plsc.get_sparse_core_info()
    nc, ns = info.num_cores, info.num_subcores
    nw = nc * ns
    jch = TOTP // (nw * 128)
    rows_w = jch * 128
    grp = 4
    mesh = plsc.VectorSubcoreMesh(core_axis_name="c", subcore_axis_name="s")

    @functools.partial(
        pl.kernel,
        mesh=mesh,
        out_type=jax.ShapeDtypeStruct((TOTP, TD), jnp.float32),
        scratch_types=[
            pltpu.VMEM((jch, 128), jnp.int32),
            pltpu.VMEM((grp * 128, TD), jnp.float32),
            pltpu.SemaphoreType.DMA,
        ],
    )
    def k(t_hbm, idx_hbm, out_hbm, idx_v, buf, sem):
        wid = lax.axis_index("s") * nc + lax.axis_index("c")
        pltpu.sync_copy(idx_hbm.at[pl.ds(wid * jch, jch)], idx_v)
        for outer in range(jch // grp):
            descs = []
            for j in range(grp):
                descs.append(pltpu.async_copy(
                    t_hbm.at[idx_v.at[outer * grp + j]],
                    buf.at[pl.ds(j * 128, 128)], sem))
            for d in descs:
                d.wait()
            pltpu.sync_copy(
                buf, out_hbm.at[pl.ds(wid * rows_w + outer * grp * 128,
                                      grp * 128)])

    return k(table, idx2d)


def _pass_c_body(g_ref, q_ref, kpm_ref, kw_ref, s8_ref, x2_ref, st_ref):
    nf = g_ref[:, 0:HID]
    pts = g_ref[:, HID:HID + 3]
    q = q_ref[...]
    qrep = jnp.broadcast_to(q[:, None, :], (MT, H, 3)).reshape(MT * H, 3)
    nb = pts - qrep
    sqn = jnp.sum(nb * nb, axis=1, keepdims=True)
    nb4 = jnp.concatenate([-2.0 * nb, jnp.ones((MT * H, 1), jnp.float32)],
                          axis=1)
    sq = lax.dot_general(nb4, kpm_ref[...],
                         (((1,), (0,)), ((), ())),
                         preferred_element_type=jnp.float32, precision=lax.Precision.HIGHEST) + sqn
    dist = jnp.sqrt(jnp.maximum(sq, 1e-12))
    nw = jnp.maximum(1.0 - dist * (1.0 / SIGMA), 0.0)
    acc = jnp.zeros((MT, HID), jnp.float32)
    for k in range(K):
        wk = (nw[:, k:k + 1] * nf).reshape(MT, H, HID).sum(axis=1)
        acc = acc + lax.dot_general(
            wk, kw_ref[k * HID:(k + 1) * HID, :],
            (((1,), (0,)), ((), ())),
            preferred_element_type=jnp.float32, precision=lax.Precision.HIGHEST)
    nfs = jnp.sum(nf, axis=1, keepdims=True)
    pos = (nfs > 0.0).astype(jnp.float32).reshape(MT, H)
    cnt = jnp.maximum(jnp.sum(pos, axis=1, keepdims=True), 1.0)
    x2 = acc / cnt
    x2_ref[...] = x2
    st_ref[0, 0:1, :] = jnp.sum(x2, axis=0, keepdims=True) @ s8_ref[...]
    st_ref[0, 1:2, :] = jnp.sum(x2 * x2, axis=0, keepdims=True) @ s8_ref[...]


def _pass_c(g, q_points, kpm, kw2d, s8):
    return pl.pallas_call(
        _pass_c_body,
        grid=(NSTEP,),
        in_specs=[
            pl.BlockSpec((MT * H, TD), lambda i: (i, 0)),
            pl.BlockSpec((MT, 3), lambda i: (i, 0)),
            pl.BlockSpec((4, K), lambda i: (0, 0)),
            pl.BlockSpec((K * HID, HID), lambda i: (0, 0)),
            pl.BlockSpec((HID, GROUPS), lambda i: (0, 0)),
        ],
        out_specs=[
            pl.BlockSpec((MT, HID), lambda i: (i, 0)),
            pl.BlockSpec((1, 2, GROUPS), lambda i: (i, 0, 0)),
        ],
        out_shape=[
            jax.ShapeDtypeStruct((N, HID), jnp.float32),
            jax.ShapeDtypeStruct((NSTEP, 2, GROUPS), jnp.float32),
        ],
    )(g, q_points, kpm, kw2d, s8)


def _pass_d_body(x2_ref, st_ref, gc_ref, bc_ref, w2_ref, r8_ref, s32_ref,
                 y_ref, st2_ref):
    tot = jnp.sum(st_ref[...], axis=0)
    scale = 1.0 / (N * (HID // GROUPS))
    mu_g = tot[0:1, :] * scale
    ms_g = tot[1:2, :] * scale
    inv_g = lax.rsqrt(ms_g - mu_g * mu_g + EPS)
    mu = mu_g @ r8_ref[...]
    inv = inv_g @ r8_ref[...]
    x2n = _leaky((x2_ref[...] - mu) * inv * gc_ref[...] + bc_ref[...])
    y = lax.dot_general(x2n, w2_ref[...],
                        (((1,), (1,)), ((), ())),
                        preferred_element_type=jnp.float32, precision=lax.Precision.HIGHEST)
    y_ref[...] = y
    st2_ref[0, 0:1, :] = jnp.sum(y, axis=0, keepdims=True) @ s32_ref[...]
    st2_ref[0, 1:2, :] = jnp.sum(y * y, axis=0, keepdims=True) @ s32_ref[...]


def _pass_d(x2, st, gc, bc, W2, r8, s32):
    return pl.pallas_call(
        _pass_d_body,
        grid=(NSTEP2,),
        in_specs=[
            pl.BlockSpec((MT2, HID), lambda i: (i, 0)),
            pl.BlockSpec((NSTEP, 2, GROUPS), lambda i: (0, 0, 0)),
            pl.BlockSpec((1, HID), lambda i: (0, 0)),
            pl.BlockSpec((1, HID), lambda i: (0, 0)),
            pl.BlockSpec((OUT_DIM, HID), lambda i: (0, 0)),
            pl.BlockSpec((GROUPS, HID), lambda i: (0, 0)),
            pl.BlockSpec((OUT_DIM, GROUPS), lambda i: (0, 0)),
        ],
        out_specs=[
            pl.BlockSpec((MT2, OUT_DIM), lambda i: (i, 0)),
            pl.BlockSpec((1, 2, GROUPS), lambda i: (i, 0, 0)),
        ],
        out_shape=[
            jax.ShapeDtypeStruct((N, OUT_DIM), jnp.float32),
            jax.ShapeDtypeStruct((NSTEP2, 2, GROUPS), jnp.float32),
        ],
    )(x2, st, gc, bc, W2, r8, s32)


def _pass_e_body(y_ref, st2_ref, g2_ref, b2_ref, sf_ref, r32_ref, o_ref):
    tot = jnp.sum(st2_ref[...], axis=0)
    scale = 1.0 / (N * (OUT_DIM // GROUPS))
    mu_g = tot[0:1, :] * scale
    ms_g = tot[1:2, :] * scale
    inv_g = lax.rsqrt(ms_g - mu_g * mu_g + EPS)
    mu = mu_g @ r32_ref[...]
    inv = inv_g @ r32_ref[...]
    yn = (y_ref[...] - mu) * inv * g2_ref[...] + b2_ref[...]
    o_ref[...] = _leaky(yn + sf_ref[...])


def _pass_e(y, st2, g2, b2, s_feats, r32):
    return pl.pallas_call(
        _pass_e_body,
        grid=(NSTEP2,),
        in_specs=[
            pl.BlockSpec((MT2, OUT_DIM), lambda i: (i, 0)),
            pl.BlockSpec((NSTEP2, 2, GROUPS), lambda i: (0, 0, 0)),
            pl.BlockSpec((1, OUT_DIM), lambda i: (0, 0)),
            pl.BlockSpec((1, OUT_DIM), lambda i: (0, 0)),
            pl.BlockSpec((MT2, OUT_DIM), lambda i: (i, 0)),
            pl.BlockSpec((GROUPS, OUT_DIM), lambda i: (0, 0)),
        ],
        out_specs=pl.BlockSpec((MT2, OUT_DIM), lambda i: (i, 0)),
        out_shape=jax.ShapeDtypeStruct((N, OUT_DIM), jnp.float32),
    )(y, st2, g2, b2, s_feats, r32)


_P64 = np.kron(np.eye(GROUPS), np.ones((8, 8))).astype(np.float32)
_S8 = np.kron(np.eye(GROUPS), np.ones((8, 1))).astype(np.float32)
_R8 = np.kron(np.eye(GROUPS), np.ones((1, 8))).astype(np.float32)
_S32 = np.kron(np.eye(GROUPS), np.ones((32, 1))).astype(np.float32)
_R32 = np.kron(np.eye(GROUPS), np.ones((1, 32))).astype(np.float32)


def kernel(s_feats, q_points, s_points, neighbor_indices, W1, gamma1, beta1,
           kernel_points, kp_weights, gamma_c, beta_c, W2, gamma2, beta2):
    idx = neighbor_indices.reshape(-1).astype(jnp.int32)
    idx = jnp.concatenate(
        [idx, jnp.zeros((TOTP - TOT,), jnp.int32)]).reshape(TOTP // 128, 128)
    kw2d = kp_weights.reshape(K * HID, HID)
    kpm = jnp.concatenate(
        [kernel_points.T, jnp.sum(kernel_points * kernel_points,
                                  axis=1)[None, :]], axis=0)

    table = _pass_a(s_feats, W1, gamma1.reshape(1, HID),
                    beta1.reshape(1, HID), s_points, _P64)
    g = _gather_rows(table, idx)
    x2, st = _pass_c(g, q_points, kpm, kw2d, _S8)
    y, st2 = _pass_d(x2, st, gamma_c.reshape(1, HID), beta_c.reshape(1, HID),
                     W2, _R8, _S32)
    return _pass_e(y, st2, gamma2.reshape(1, OUT_DIM),
                   beta2.reshape(1, OUT_DIM), s_feats, _R32)
